# baseline (speedup 1.0000x reference)
import functools

import jax
import jax.numpy as jnp
from jax import lax
from jax.experimental import pallas as pl
from jax.experimental.pallas import tpu as pltpu
from jax.experimental.pallas import tpu_sc as plsc

EMBED_DIM = 64
NUM_CORES = 2
NUM_SUBCORES = 16
NUM_WORKERS = NUM_CORES * NUM_SUBCORES
NBUF = 8


@functools.partial(jax.jit, static_argnames=("rows_per_w", "hist"))
def _sc_gather(x, table, *, rows_per_w, hist):
    batch = NUM_WORKERS * rows_per_w
    mesh = plsc.VectorSubcoreMesh(core_axis_name="c", subcore_axis_name="s")

    @functools.partial(
        pl.kernel,
        mesh=mesh,
        compiler_params=pltpu.CompilerParams(use_tc_tiling_on_sc=False),
        out_type=jax.ShapeDtypeStruct((batch, hist, EMBED_DIM), jnp.float32),
        scratch_types=[
            pltpu.VMEM((rows_per_w, hist), jnp.int32),
            pltpu.VMEM((NBUF, hist, EMBED_DIM), jnp.float32),
        ] + [pltpu.SemaphoreType.DMA] * NBUF,
    )
    def k(x_hbm, table_hbm, out_hbm, idx_v, rows_v, *sems):
        wid = lax.axis_index("s") * NUM_CORES + lax.axis_index("c")
        base = wid * rows_per_w
        pltpu.sync_copy(x_hbm.at[pl.ds(base, rows_per_w)], idx_v)

        def start(i, b):
            pltpu.async_copy(table_hbm.at[idx_v.at[i]], rows_v.at[b], sems[b])

        def drain(i, b):
            pltpu.make_async_copy(
                table_hbm.at[pl.ds(0, hist)], rows_v.at[b], sems[b]
            ).wait()
            pltpu.sync_copy(rows_v.at[b], out_hbm.at[base + i])

        for b in range(NBUF):
            start(b, b)

        def outer(j, carry):
            i0 = j * NBUF
            for b in range(NBUF):
                drain(i0 + b, b)
                start(i0 + b + NBUF, b)
            return carry

        lax.fori_loop(0, rows_per_w // NBUF - 1, outer, 0)
        for b in range(NBUF):
            drain(rows_per_w - NBUF + b, b)

    return k(x, table)


def kernel(x, table):
    batch, hist = x.shape
    return _sc_gather(
        x.astype(jnp.int32), table,
        rows_per_w=batch // NUM_WORKERS, hist=hist,
    )
